# baseline (speedup 1.0000x reference)
import functools

import numpy as np

import jax
import jax.numpy as jnp
from jax import lax
from jax.experimental import pallas as pl
from jax.experimental.pallas import tpu as pltpu
from jax.experimental.pallas import tpu_sc as plsc

N_SC = 2
N_TILE = 16
N_WORKER = N_SC * N_TILE
CHUNK = 128


def _make_sc_agg(n_pad, d, e_pad, nbuf, chunk=CHUNK, tc_tiling=False):
    per_tile = e_pad // N_WORKER
    n_chunks = per_tile // chunk
    n_grp = n_chunks // nbuf
    assert per_tile % chunk == 0 and n_chunks % nbuf == 0
    assert (n_chunks // nbuf) % 2 == 0
    rows_per_tile = n_pad // N_TILE
    assert n_pad % N_TILE == 0

    mesh = plsc.VectorSubcoreMesh(core_axis_name="c", subcore_axis_name="s")

    @functools.partial(
        pl.kernel,
        mesh=mesh,
        out_type=(
            jax.ShapeDtypeStruct((n_pad, d), jnp.float32),
            jax.ShapeDtypeStruct((n_pad, d), jnp.float32),
        ),
        compiler_params=pltpu.CompilerParams(use_tc_tiling_on_sc=tc_tiling),
        scratch_types=[
            pltpu.VMEM_SHARED((n_pad, d), jnp.float32),
            pltpu.VMEM((2, nbuf, chunk), jnp.int32),
            pltpu.VMEM((2, nbuf, chunk), jnp.int32),
            pltpu.VMEM((nbuf, chunk, d), jnp.float32),
            pltpu.SemaphoreType.DMA((nbuf,)),
            pltpu.SemaphoreType.DMA((nbuf,)),
            pltpu.SemaphoreType.DMA((2,)),
            pltpu.SemaphoreType.DMA((2,)),
        ],
    )
    def k(src3, dst3, y_hbm, zeros_hbm, out0, out1, z_sp, sidx, didx,
          rows, gsem, ssem, isem_s, isem_d):
        c = lax.axis_index("c")
        s = lax.axis_index("s")
        wid = s * N_SC + c
        row0 = s * rows_per_tile
        pltpu.sync_copy(zeros_hbm.at[pl.ds(row0, rows_per_tile)],
                        z_sp.at[pl.ds(row0, rows_per_tile)])

        gbase = wid * n_grp
        pltpu.async_copy(src3.at[gbase], sidx.at[0], isem_s.at[0])
        pltpu.async_copy(dst3.at[gbase], didx.at[0], isem_d.at[0])
        plsc.subcore_barrier()

        @pl.loop(0, n_grp // 2)
        def _grp2(gg):
            for p in range(2):
                grp = gg * 2 + p
                nxt = grp + 1

                pltpu.make_async_copy(src3.at[gbase + grp], sidx.at[p],
                                      isem_s.at[p]).wait()
                pltpu.make_async_copy(dst3.at[gbase + grp], didx.at[p],
                                      isem_d.at[p]).wait()
                gds = []
                for b in range(nbuf):
                    @pl.when(grp > 0)
                    def _():
                        pltpu.make_async_copy(
                            rows.at[b], z_sp.at[didx.at[p].at[b]],
                            ssem.at[b]).wait()

                    gds.append(pltpu.async_copy(y_hbm.at[sidx.at[p].at[b]],
                                                rows.at[b], gsem.at[b]))

                @pl.when(nxt < n_grp)
                def _():
                    pltpu.async_copy(src3.at[gbase + nxt], sidx.at[1 - p],
                                     isem_s.at[1 - p])
                    pltpu.async_copy(dst3.at[gbase + nxt], didx.at[1 - p],
                                     isem_d.at[1 - p])

                for b in range(nbuf):
                    gds[b].wait()
                    pltpu.async_copy(rows.at[b], z_sp.at[didx.at[p].at[b]],
                                     ssem.at[b], add=True)

        lastp = (n_grp - 1) % 2
        for b in range(nbuf):
            pltpu.make_async_copy(rows.at[b], z_sp.at[didx.at[lastp].at[b]],
                                  ssem.at[b]).wait()

        plsc.subcore_barrier()

        @pl.when(c == 0)
        def _():
            pltpu.sync_copy(z_sp.at[pl.ds(row0, rows_per_tile)],
                            out0.at[pl.ds(row0, rows_per_tile)])

        @pl.when(c == 1)
        def _():
            pltpu.sync_copy(z_sp.at[pl.ds(row0, rows_per_tile)],
                            out1.at[pl.ds(row0, rows_per_tile)])

    return k


def _make_sc_deg(n_pad, e_pad, nbuf):
    per_tile = e_pad // N_WORKER
    n_chunks = per_tile // CHUNK
    n_grp = n_chunks // nbuf
    rows_per_tile = n_pad // N_TILE
    d = 16

    mesh = plsc.VectorSubcoreMesh(core_axis_name="c", subcore_axis_name="s")

    @functools.partial(
        pl.kernel,
        mesh=mesh,
        out_type=(
            jax.ShapeDtypeStruct((n_pad, d), jnp.float32),
            jax.ShapeDtypeStruct((n_pad, d), jnp.float32),
        ),
        compiler_params=pltpu.CompilerParams(use_tc_tiling_on_sc=False),
        scratch_types=[
            pltpu.VMEM_SHARED((n_pad, d), jnp.float32),
            pltpu.VMEM((2, nbuf, CHUNK), jnp.int32),
            pltpu.VMEM((CHUNK, d), jnp.float32),
            pltpu.SemaphoreType.DMA((nbuf,)),
            pltpu.SemaphoreType.DMA((2,)),
        ],
    )
    def k(dst3, ones_hbm, zeros_hbm, out0, out1, z_sp, didx, ones_v, ssem,
          isem):
        c = lax.axis_index("c")
        s = lax.axis_index("s")
        wid = s * N_SC + c
        row0 = s * rows_per_tile
        pltpu.sync_copy(zeros_hbm.at[pl.ds(row0, rows_per_tile)],
                        z_sp.at[pl.ds(row0, rows_per_tile)])
        pltpu.sync_copy(ones_hbm, ones_v)

        gbase = wid * n_grp
        pltpu.async_copy(dst3.at[gbase], didx.at[0], isem.at[0])
        plsc.subcore_barrier()

        @pl.loop(0, n_grp // 2)
        def _grp2(gg):
            for p in range(2):
                grp = gg * 2 + p
                nxt = grp + 1

                @pl.when(nxt < n_grp)
                def _():
                    pltpu.async_copy(dst3.at[gbase + nxt], didx.at[1 - p],
                                     isem.at[1 - p])

                pltpu.make_async_copy(dst3.at[gbase + grp], didx.at[p],
                                      isem.at[p]).wait()
                sds = [pltpu.async_copy(ones_v, z_sp.at[didx.at[p].at[b]],
                                        ssem.at[b], add=True)
                       for b in range(nbuf)]
                for dsc in sds:
                    dsc.wait()

        plsc.subcore_barrier()

        @pl.when(c == 0)
        def _():
            pltpu.sync_copy(z_sp.at[pl.ds(row0, rows_per_tile)],
                            out0.at[pl.ds(row0, rows_per_tile)])

        @pl.when(c == 1)
        def _():
            pltpu.sync_copy(z_sp.at[pl.ds(row0, rows_per_tile)],
                            out1.at[pl.ds(row0, rows_per_tile)])

    return k


_ROWS = 1024


def _row_grid(n_pad):
    return (n_pad + _ROWS - 1) // _ROWS


def _rb(d):
    return pl.BlockSpec((_ROWS, d), lambda i: (i, 0))


def _full(shape):
    return pl.BlockSpec(shape, lambda i: tuple(0 for _ in shape))


def _tc1_body(d0, d1, x, dinv_o, y1_o):
    deg = d0[...][:, 0:1] + d1[...][:, 0:1] + 1.0
    dinv = lax.rsqrt(deg)
    dinv_o[...] = dinv
    y1_o[...] = x[...] * dinv


def _tc1(d0, d1, x, n_pad, d_in):
    return pl.pallas_call(
        _tc1_body,
        grid=(_row_grid(n_pad),),
        in_specs=[_rb(16), _rb(16), _rb(d_in)],
        out_specs=[_rb(1), _rb(d_in)],
        out_shape=[
            jax.ShapeDtypeStruct((n_pad, 1), jnp.float32),
            jax.ShapeDtypeStruct((n_pad, d_in), jnp.float32),
        ],
    )(d0, d1, x)


def _tc2_body(z0, z1, x, dinv_r, w1, b1, w2, t2_o, y2_o):
    dinv = dinv_r[...]
    agg = (z0[...] + z1[...]) * dinv + x[...] * (dinv * dinv)
    h1 = jnp.dot(agg, w1[...], preferred_element_type=jnp.float32) + b1[...]
    h1 = jnp.maximum(h1, 0.0)
    t2 = jnp.dot(h1, w2[...], preferred_element_type=jnp.float32)
    t2_o[...] = t2
    y2_o[...] = t2 * dinv


def _tc2(z0, z1, x, dinv, w1, b1, w2, n_pad):
    return pl.pallas_call(
        _tc2_body,
        grid=(_row_grid(n_pad),),
        in_specs=[_rb(128), _rb(128), _rb(128), _rb(1),
                  _full((128, 256)), _full((1, 256)), _full((256, 16))],
        out_specs=[_rb(16), _rb(16)],
        out_shape=[
            jax.ShapeDtypeStruct((n_pad, 16), jnp.float32),
            jax.ShapeDtypeStruct((n_pad, 16), jnp.float32),
        ],
    )(z0, z1, x, dinv, w1, b1, w2)


def _tc3_body(z0, z1, t2, dinv_r, b2, w3, t3_o, y3_o):
    dinv = dinv_r[...]
    agg = (z0[...] + z1[...]) * dinv + t2[...] * (dinv * dinv)
    h2 = jnp.maximum(agg + b2[...], 0.0)
    t3 = jnp.dot(h2, w3[...], preferred_element_type=jnp.float32)
    t3_o[...] = t3
    y3_o[...] = t3 * dinv


def _tc3(z0, z1, t2, dinv, b2, w3, n_pad):
    return pl.pallas_call(
        _tc3_body,
        grid=(_row_grid(n_pad),),
        in_specs=[_rb(16), _rb(16), _rb(16), _rb(1),
                  _full((1, 16)), _full((16, 2))],
        out_specs=[_rb(2), _rb(2)],
        out_shape=[
            jax.ShapeDtypeStruct((n_pad, 2), jnp.float32),
            jax.ShapeDtypeStruct((n_pad, 2), jnp.float32),
        ],
    )(z0, z1, t2, dinv, b2, w3)


def _tc4_body(z0, z1, t3, dinv_r, b3, out_o):
    dinv = dinv_r[...]
    z = (z0[...] + z1[...])[:, 0:2]
    agg = z * dinv + t3[...] * (dinv * dinv)
    out_o[...] = jax.nn.sigmoid(agg + b3[...])


def _tc4(z0, z1, t3, dinv, b3, n_pad):
    return pl.pallas_call(
        _tc4_body,
        grid=(_row_grid(n_pad),),
        in_specs=[_rb(16), _rb(16), _rb(2), _rb(1), _full((1, 2))],
        out_specs=_rb(2),
        out_shape=jax.ShapeDtypeStruct((n_pad, 2), jnp.float32),
    )(z0, z1, t3, dinv, b3)


def kernel(x, edge_index, W1, b1, W2, b2, W3, b3):
    n = x.shape[0]
    d_in = x.shape[1]
    e = edge_index.shape[1]

    n_pad = ((n + 16 + 127) // 128) * 128
    e_quant = N_WORKER * CHUNK * 8
    e_pad = ((e + e_quant - 1) // e_quant) * e_quant

    src = edge_index[0].astype(jnp.int32)
    dst = edge_index[1].astype(jnp.int32)
    pad = e_pad - e
    if pad:
        pad_src = (np.arange(pad, dtype=np.int32) * 37) % n
        pad_dst = (n + (np.arange(pad, dtype=np.int32) % 16)).astype(np.int32)
        src = jnp.concatenate([src, pad_src])
        dst = jnp.concatenate([dst, pad_dst])

    zeros_wide = np.zeros((n_pad, 128), np.float32)
    zeros_16 = np.zeros((n_pad, 16), np.float32)
    ones_16 = np.ones((CHUNK, 16), np.float32)

    ck_wide = 128
    nb_wide = 2
    nb_thin = 10
    src2 = src.reshape(-1, nb_wide, ck_wide)
    dst2 = dst.reshape(-1, nb_wide, ck_wide)
    src8 = src.reshape(-1, nb_thin, CHUNK)
    dst8 = dst.reshape(-1, nb_thin, CHUNK)

    d0, d1 = _make_sc_deg(n_pad, e_pad, nb_thin)(dst8, ones_16, zeros_16)
    dinv, y1 = _tc1(d0, d1, x, n_pad, d_in)

    z0, z1 = _make_sc_agg(n_pad, d_in, e_pad, nb_wide, ck_wide, tc_tiling=True)(
        src2, dst2, y1, zeros_wide)
    t2, y2 = _tc2(z0, z1, x, dinv, W1, b1.reshape(1, -1), W2, n_pad)

    u0, u1 = _make_sc_agg(n_pad, 16, e_pad, nb_thin)(src8, dst8, y2, zeros_16)
    t3, y3 = _tc3(u0, u1, t2, dinv, b2.reshape(1, -1), W3, n_pad)

    y3_p = jnp.pad(y3, ((0, 0), (0, 14)))
    v0, v1 = _make_sc_agg(n_pad, 16, e_pad, nb_thin)(src8, dst8, y3_p, zeros_16)
    out = _tc4(v0, v1, t3, dinv, b3.reshape(1, -1), n_pad)

    return out[:n]


# speedup vs baseline: 1.1048x; 1.1048x over previous
import functools

import numpy as np

import jax
import jax.numpy as jnp
from jax import lax
from jax.experimental import pallas as pl
from jax.experimental.pallas import tpu as pltpu
from jax.experimental.pallas import tpu_sc as plsc

N_SC = 2
N_TILE = 16
N_WORKER = N_SC * N_TILE
CHUNK = 128


def _make_sc_agg(n_pad, d, e_pad, nbuf, chunk=CHUNK, tc_tiling=False):
    per_tile = e_pad // N_WORKER
    n_chunks = per_tile // chunk
    n_grp = n_chunks // nbuf
    assert per_tile % chunk == 0 and n_chunks % nbuf == 0
    assert (n_chunks // nbuf) % 2 == 0
    rows_per_tile = n_pad // N_TILE
    assert n_pad % N_TILE == 0

    mesh = plsc.VectorSubcoreMesh(core_axis_name="c", subcore_axis_name="s")

    @functools.partial(
        pl.kernel,
        mesh=mesh,
        out_type=(
            jax.ShapeDtypeStruct((n_pad, d), jnp.float32),
            jax.ShapeDtypeStruct((n_pad, d), jnp.float32),
        ),
        compiler_params=pltpu.CompilerParams(use_tc_tiling_on_sc=tc_tiling),
        scratch_types=[
            pltpu.VMEM_SHARED((n_pad, d), jnp.float32),
            pltpu.VMEM((2, nbuf, chunk), jnp.int32),
            pltpu.VMEM((2, nbuf, chunk), jnp.int32),
            pltpu.VMEM((nbuf, chunk, d), jnp.float32),
            pltpu.SemaphoreType.DMA((nbuf,)),
            pltpu.SemaphoreType.DMA((nbuf,)),
            pltpu.SemaphoreType.DMA((2,)),
            pltpu.SemaphoreType.DMA((2,)),
        ],
    )
    def k(src3, dst3, y_hbm, zeros_hbm, out0, out1, z_sp, sidx, didx,
          rows, gsem, ssem, isem_s, isem_d):
        c = lax.axis_index("c")
        s = lax.axis_index("s")
        wid = s * N_SC + c
        row0 = s * rows_per_tile
        pltpu.sync_copy(zeros_hbm.at[pl.ds(row0, rows_per_tile)],
                        z_sp.at[pl.ds(row0, rows_per_tile)])

        gbase = wid * n_grp
        pltpu.async_copy(src3.at[gbase], sidx.at[0], isem_s.at[0])
        pltpu.async_copy(dst3.at[gbase], didx.at[0], isem_d.at[0])
        plsc.subcore_barrier()

        @pl.loop(0, n_grp // 2)
        def _grp2(gg):
            for p in range(2):
                grp = gg * 2 + p
                nxt = grp + 1

                pltpu.make_async_copy(src3.at[gbase + grp], sidx.at[p],
                                      isem_s.at[p]).wait()
                pltpu.make_async_copy(dst3.at[gbase + grp], didx.at[p],
                                      isem_d.at[p]).wait()
                gds = []
                for b in range(nbuf):
                    @pl.when(grp > 0)
                    def _():
                        pltpu.make_async_copy(
                            rows.at[b], z_sp.at[didx.at[p].at[b]],
                            ssem.at[b]).wait()

                    gds.append(pltpu.async_copy(y_hbm.at[sidx.at[p].at[b]],
                                                rows.at[b], gsem.at[b]))

                @pl.when(nxt < n_grp)
                def _():
                    pltpu.async_copy(src3.at[gbase + nxt], sidx.at[1 - p],
                                     isem_s.at[1 - p])
                    pltpu.async_copy(dst3.at[gbase + nxt], didx.at[1 - p],
                                     isem_d.at[1 - p])

                for b in range(nbuf):
                    gds[b].wait()
                    pltpu.async_copy(rows.at[b], z_sp.at[didx.at[p].at[b]],
                                     ssem.at[b], add=True)

        lastp = (n_grp - 1) % 2
        for b in range(nbuf):
            pltpu.make_async_copy(rows.at[b], z_sp.at[didx.at[lastp].at[b]],
                                  ssem.at[b]).wait()

        plsc.subcore_barrier()

        @pl.when(c == 0)
        def _():
            pltpu.sync_copy(z_sp.at[pl.ds(row0, rows_per_tile)],
                            out0.at[pl.ds(row0, rows_per_tile)])

        @pl.when(c == 1)
        def _():
            pltpu.sync_copy(z_sp.at[pl.ds(row0, rows_per_tile)],
                            out1.at[pl.ds(row0, rows_per_tile)])

    return k


def _make_sc_deg(n_pad, e_pad, nbuf):
    per_tile = e_pad // N_WORKER
    n_chunks = per_tile // CHUNK
    n_grp = n_chunks // nbuf
    rows_per_tile = n_pad // N_TILE
    d = 16

    mesh = plsc.VectorSubcoreMesh(core_axis_name="c", subcore_axis_name="s")

    @functools.partial(
        pl.kernel,
        mesh=mesh,
        out_type=(
            jax.ShapeDtypeStruct((n_pad, d), jnp.float32),
            jax.ShapeDtypeStruct((n_pad, d), jnp.float32),
        ),
        compiler_params=pltpu.CompilerParams(use_tc_tiling_on_sc=False),
        scratch_types=[
            pltpu.VMEM_SHARED((n_pad, d), jnp.float32),
            pltpu.VMEM((2, nbuf, CHUNK), jnp.int32),
            pltpu.VMEM((CHUNK, d), jnp.float32),
            pltpu.SemaphoreType.DMA((nbuf,)),
            pltpu.SemaphoreType.DMA((2,)),
        ],
    )
    def k(dst3, ones_hbm, zeros_hbm, out0, out1, z_sp, didx, ones_v, ssem,
          isem):
        c = lax.axis_index("c")
        s = lax.axis_index("s")
        wid = s * N_SC + c
        row0 = s * rows_per_tile
        pltpu.sync_copy(zeros_hbm.at[pl.ds(row0, rows_per_tile)],
                        z_sp.at[pl.ds(row0, rows_per_tile)])
        pltpu.sync_copy(ones_hbm, ones_v)

        gbase = wid * n_grp
        pltpu.async_copy(dst3.at[gbase], didx.at[0], isem.at[0])
        plsc.subcore_barrier()

        @pl.loop(0, n_grp // 2)
        def _grp2(gg):
            for p in range(2):
                grp = gg * 2 + p
                nxt = grp + 1

                @pl.when(nxt < n_grp)
                def _():
                    pltpu.async_copy(dst3.at[gbase + nxt], didx.at[1 - p],
                                     isem.at[1 - p])

                pltpu.make_async_copy(dst3.at[gbase + grp], didx.at[p],
                                      isem.at[p]).wait()
                sds = [pltpu.async_copy(ones_v, z_sp.at[didx.at[p].at[b]],
                                        ssem.at[b], add=True)
                       for b in range(nbuf)]
                for dsc in sds:
                    dsc.wait()

        plsc.subcore_barrier()

        @pl.when(c == 0)
        def _():
            pltpu.sync_copy(z_sp.at[pl.ds(row0, rows_per_tile)],
                            out0.at[pl.ds(row0, rows_per_tile)])

        @pl.when(c == 1)
        def _():
            pltpu.sync_copy(z_sp.at[pl.ds(row0, rows_per_tile)],
                            out1.at[pl.ds(row0, rows_per_tile)])

    return k


_ROWS = 1024


def _row_grid(n_pad):
    return (n_pad + _ROWS - 1) // _ROWS


def _rb(d):
    return pl.BlockSpec((_ROWS, d), lambda i: (i, 0))


def _full(shape):
    return pl.BlockSpec(shape, lambda i: tuple(0 for _ in shape))


def _tc1_body(d0, d1, x, dinv_o, y1_o):
    deg = d0[...][:, 0:1] + d1[...][:, 0:1] + 1.0
    dinv = lax.rsqrt(deg)
    dinv_o[...] = dinv
    y1_o[...] = x[...] * dinv


def _tc1(d0, d1, x, n_pad, d_in):
    return pl.pallas_call(
        _tc1_body,
        grid=(_row_grid(n_pad),),
        in_specs=[_rb(16), _rb(16), _rb(d_in)],
        out_specs=[_rb(1), _rb(d_in)],
        out_shape=[
            jax.ShapeDtypeStruct((n_pad, 1), jnp.float32),
            jax.ShapeDtypeStruct((n_pad, d_in), jnp.float32),
        ],
    )(d0, d1, x)


def _tc2_body(z0, z1, x, dinv_r, w1, b1, w2, t2_o, y2_o):
    dinv = dinv_r[...]
    agg = (z0[...] + z1[...]) * dinv + x[...] * (dinv * dinv)
    h1 = jnp.dot(agg, w1[...], preferred_element_type=jnp.float32) + b1[...]
    h1 = jnp.maximum(h1, 0.0)
    t2 = jnp.dot(h1, w2[...], preferred_element_type=jnp.float32)
    t2_o[...] = t2
    y2_o[...] = t2 * dinv


def _tc2(z0, z1, x, dinv, w1, b1, w2, n_pad):
    return pl.pallas_call(
        _tc2_body,
        grid=(_row_grid(n_pad),),
        in_specs=[_rb(128), _rb(128), _rb(128), _rb(1),
                  _full((128, 256)), _full((1, 256)), _full((256, 16))],
        out_specs=[_rb(16), _rb(16)],
        out_shape=[
            jax.ShapeDtypeStruct((n_pad, 16), jnp.float32),
            jax.ShapeDtypeStruct((n_pad, 16), jnp.float32),
        ],
    )(z0, z1, x, dinv, w1, b1, w2)


def _tc3_body(z0, z1, t2, dinv_r, b2, w3, t3_o, y3_o):
    dinv = dinv_r[...]
    agg = (z0[...] + z1[...]) * dinv + t2[...] * (dinv * dinv)
    h2 = jnp.maximum(agg + b2[...], 0.0)
    t3 = jnp.dot(h2, w3[...], preferred_element_type=jnp.float32)
    t3_o[...] = t3
    y3_o[...] = t3 * dinv


def _tc3(z0, z1, t2, dinv, b2, w3, n_pad):
    return pl.pallas_call(
        _tc3_body,
        grid=(_row_grid(n_pad),),
        in_specs=[_rb(16), _rb(16), _rb(16), _rb(1),
                  _full((1, 16)), _full((16, 2))],
        out_specs=[_rb(2), _rb(2)],
        out_shape=[
            jax.ShapeDtypeStruct((n_pad, 2), jnp.float32),
            jax.ShapeDtypeStruct((n_pad, 2), jnp.float32),
        ],
    )(z0, z1, t2, dinv, b2, w3)


def _tc4_body(z0, z1, t3, dinv_r, b3, out_o):
    dinv = dinv_r[...]
    z = (z0[...] + z1[...])[:, 0:2]
    agg = z * dinv + t3[...] * (dinv * dinv)
    out_o[...] = jax.nn.sigmoid(agg + b3[...])


def _tc4(z0, z1, t3, dinv, b3, n_pad):
    return pl.pallas_call(
        _tc4_body,
        grid=(_row_grid(n_pad),),
        in_specs=[_rb(16), _rb(16), _rb(2), _rb(1), _full((1, 2))],
        out_specs=_rb(2),
        out_shape=jax.ShapeDtypeStruct((n_pad, 2), jnp.float32),
    )(z0, z1, t3, dinv, b3)


def kernel(x, edge_index, W1, b1, W2, b2, W3, b3):
    n = x.shape[0]
    d_in = x.shape[1]
    e = edge_index.shape[1]

    n_pad = ((n + 16 + 127) // 128) * 128
    e_quant = N_WORKER * CHUNK * 8
    e_pad = ((e + e_quant - 1) // e_quant) * e_quant

    src = edge_index[0].astype(jnp.int32)
    dst = edge_index[1].astype(jnp.int32)
    pad = e_pad - e
    if pad:
        pad_src = (np.arange(pad, dtype=np.int32) * 37) % n
        pad_dst = (n + (np.arange(pad, dtype=np.int32) % 16)).astype(np.int32)
        src = jnp.concatenate([src, pad_src])
        dst = jnp.concatenate([dst, pad_dst])

    zeros_wide = np.zeros((n_pad, 128), np.float32)
    zeros_16 = np.zeros((n_pad, 16), np.float32)
    ones_16 = np.ones((CHUNK, 16), np.float32)

    ck_wide = 64
    nb_wide = 5
    nb_thin = 10
    src2 = src.reshape(-1, nb_wide, ck_wide)
    dst2 = dst.reshape(-1, nb_wide, ck_wide)
    src8 = src.reshape(-1, nb_thin, CHUNK)
    dst8 = dst.reshape(-1, nb_thin, CHUNK)

    d0, d1 = _make_sc_deg(n_pad, e_pad, nb_thin)(dst8, ones_16, zeros_16)
    dinv, y1 = _tc1(d0, d1, x, n_pad, d_in)

    z0, z1 = _make_sc_agg(n_pad, d_in, e_pad, nb_wide, ck_wide, tc_tiling=True)(
        src2, dst2, y1, zeros_wide)
    t2, y2 = _tc2(z0, z1, x, dinv, W1, b1.reshape(1, -1), W2, n_pad)

    u0, u1 = _make_sc_agg(n_pad, 16, e_pad, nb_thin)(src8, dst8, y2, zeros_16)
    t3, y3 = _tc3(u0, u1, t2, dinv, b2.reshape(1, -1), W3, n_pad)

    y3_p = jnp.pad(y3, ((0, 0), (0, 14)))
    v0, v1 = _make_sc_agg(n_pad, 16, e_pad, nb_thin)(src8, dst8, y3_p, zeros_16)
    out = _tc4(v0, v1, t3, dinv, b3.reshape(1, -1), n_pad)

    return out[:n]


# speedup vs baseline: 1.1049x; 1.0001x over previous
import functools

import numpy as np

import jax
import jax.numpy as jnp
from jax import lax
from jax.experimental import pallas as pl
from jax.experimental.pallas import tpu as pltpu
from jax.experimental.pallas import tpu_sc as plsc

N_SC = 2
N_TILE = 16
N_WORKER = N_SC * N_TILE
CHUNK = 128


def _make_sc_agg(n_pad, d, e_pad, nbuf, chunk=CHUNK, tc_tiling=False):
    per_tile = e_pad // N_WORKER
    n_chunks = per_tile // chunk
    n_grp = n_chunks // nbuf
    assert per_tile % chunk == 0 and n_chunks % nbuf == 0
    assert (n_chunks // nbuf) % 2 == 0
    rows_per_tile = n_pad // N_TILE
    assert n_pad % N_TILE == 0

    mesh = plsc.VectorSubcoreMesh(core_axis_name="c", subcore_axis_name="s")

    @functools.partial(
        pl.kernel,
        mesh=mesh,
        out_type=(
            jax.ShapeDtypeStruct((n_pad, d), jnp.float32),
            jax.ShapeDtypeStruct((n_pad, d), jnp.float32),
        ),
        compiler_params=pltpu.CompilerParams(use_tc_tiling_on_sc=tc_tiling),
        scratch_types=[
            pltpu.VMEM_SHARED((n_pad, d), jnp.float32),
            pltpu.VMEM((2, nbuf, chunk), jnp.int32),
            pltpu.VMEM((2, nbuf, chunk), jnp.int32),
            pltpu.VMEM((nbuf, chunk, d), jnp.float32),
            pltpu.SemaphoreType.DMA((nbuf,)),
            pltpu.SemaphoreType.DMA((nbuf,)),
            pltpu.SemaphoreType.DMA((2,)),
            pltpu.SemaphoreType.DMA((2,)),
        ],
    )
    def k(src3, dst3, y_hbm, zeros_hbm, out0, out1, z_sp, sidx, didx,
          rows, gsem, ssem, isem_s, isem_d):
        c = lax.axis_index("c")
        s = lax.axis_index("s")
        wid = s * N_SC + c
        row0 = s * rows_per_tile
        pltpu.sync_copy(zeros_hbm.at[pl.ds(row0, rows_per_tile)],
                        z_sp.at[pl.ds(row0, rows_per_tile)])

        gbase = wid * n_grp
        pltpu.async_copy(src3.at[gbase], sidx.at[0], isem_s.at[0])
        pltpu.async_copy(dst3.at[gbase], didx.at[0], isem_d.at[0])
        plsc.subcore_barrier()

        @pl.loop(0, n_grp // 2)
        def _grp2(gg):
            for p in range(2):
                grp = gg * 2 + p
                nxt = grp + 1

                pltpu.make_async_copy(src3.at[gbase + grp], sidx.at[p],
                                      isem_s.at[p]).wait()
                pltpu.make_async_copy(dst3.at[gbase + grp], didx.at[p],
                                      isem_d.at[p]).wait()
                gds = []
                for b in range(nbuf):
                    @pl.when(grp > 0)
                    def _():
                        pltpu.make_async_copy(
                            rows.at[b], z_sp.at[didx.at[p].at[b]],
                            ssem.at[b]).wait()

                    gds.append(pltpu.async_copy(y_hbm.at[sidx.at[p].at[b]],
                                                rows.at[b], gsem.at[b]))

                @pl.when(nxt < n_grp)
                def _():
                    pltpu.async_copy(src3.at[gbase + nxt], sidx.at[1 - p],
                                     isem_s.at[1 - p])
                    pltpu.async_copy(dst3.at[gbase + nxt], didx.at[1 - p],
                                     isem_d.at[1 - p])

                for b in range(nbuf):
                    gds[b].wait()
                    pltpu.async_copy(rows.at[b], z_sp.at[didx.at[p].at[b]],
                                     ssem.at[b], add=True)

        lastp = (n_grp - 1) % 2
        for b in range(nbuf):
            pltpu.make_async_copy(rows.at[b], z_sp.at[didx.at[lastp].at[b]],
                                  ssem.at[b]).wait()

        plsc.subcore_barrier()

        @pl.when(c == 0)
        def _():
            pltpu.sync_copy(z_sp.at[pl.ds(row0, rows_per_tile)],
                            out0.at[pl.ds(row0, rows_per_tile)])

        @pl.when(c == 1)
        def _():
            pltpu.sync_copy(z_sp.at[pl.ds(row0, rows_per_tile)],
                            out1.at[pl.ds(row0, rows_per_tile)])

    return k


def _make_sc_deg(n_pad, e_pad, nbuf):
    per_tile = e_pad // N_WORKER
    n_chunks = per_tile // CHUNK
    n_grp = n_chunks // nbuf
    rows_per_tile = n_pad // N_TILE
    d = 16

    mesh = plsc.VectorSubcoreMesh(core_axis_name="c", subcore_axis_name="s")

    @functools.partial(
        pl.kernel,
        mesh=mesh,
        out_type=(
            jax.ShapeDtypeStruct((n_pad, d), jnp.float32),
            jax.ShapeDtypeStruct((n_pad, d), jnp.float32),
        ),
        compiler_params=pltpu.CompilerParams(use_tc_tiling_on_sc=False),
        scratch_types=[
            pltpu.VMEM_SHARED((n_pad, d), jnp.float32),
            pltpu.VMEM((2, nbuf, CHUNK), jnp.int32),
            pltpu.VMEM((CHUNK, d), jnp.float32),
            pltpu.SemaphoreType.DMA((nbuf,)),
            pltpu.SemaphoreType.DMA((2,)),
        ],
    )
    def k(dst3, ones_hbm, zeros_hbm, out0, out1, z_sp, didx, ones_v, ssem,
          isem):
        c = lax.axis_index("c")
        s = lax.axis_index("s")
        wid = s * N_SC + c
        row0 = s * rows_per_tile
        pltpu.sync_copy(zeros_hbm.at[pl.ds(row0, rows_per_tile)],
                        z_sp.at[pl.ds(row0, rows_per_tile)])
        pltpu.sync_copy(ones_hbm, ones_v)

        gbase = wid * n_grp
        pltpu.async_copy(dst3.at[gbase], didx.at[0], isem.at[0])
        plsc.subcore_barrier()

        @pl.loop(0, n_grp // 2)
        def _grp2(gg):
            for p in range(2):
                grp = gg * 2 + p
                nxt = grp + 1

                @pl.when(nxt < n_grp)
                def _():
                    pltpu.async_copy(dst3.at[gbase + nxt], didx.at[1 - p],
                                     isem.at[1 - p])

                pltpu.make_async_copy(dst3.at[gbase + grp], didx.at[p],
                                      isem.at[p]).wait()
                sds = [pltpu.async_copy(ones_v, z_sp.at[didx.at[p].at[b]],
                                        ssem.at[b], add=True)
                       for b in range(nbuf)]
                for dsc in sds:
                    dsc.wait()

        plsc.subcore_barrier()

        @pl.when(c == 0)
        def _():
            pltpu.sync_copy(z_sp.at[pl.ds(row0, rows_per_tile)],
                            out0.at[pl.ds(row0, rows_per_tile)])

        @pl.when(c == 1)
        def _():
            pltpu.sync_copy(z_sp.at[pl.ds(row0, rows_per_tile)],
                            out1.at[pl.ds(row0, rows_per_tile)])

    return k


_ROWS = 1024


def _row_grid(n_pad):
    return (n_pad + _ROWS - 1) // _ROWS


def _rb(d):
    return pl.BlockSpec((_ROWS, d), lambda i: (i, 0))


def _full(shape):
    return pl.BlockSpec(shape, lambda i: tuple(0 for _ in shape))


def _tc1_body(d0, d1, x, dinv_o, y1_o):
    deg = d0[...][:, 0:1] + d1[...][:, 0:1] + 1.0
    dinv = lax.rsqrt(deg)
    dinv_o[...] = dinv
    y1_o[...] = x[...] * dinv


def _tc1(d0, d1, x, n_pad, d_in):
    return pl.pallas_call(
        _tc1_body,
        grid=(_row_grid(n_pad),),
        in_specs=[_rb(16), _rb(16), _rb(d_in)],
        out_specs=[_rb(1), _rb(d_in)],
        out_shape=[
            jax.ShapeDtypeStruct((n_pad, 1), jnp.float32),
            jax.ShapeDtypeStruct((n_pad, d_in), jnp.float32),
        ],
    )(d0, d1, x)


def _tc2_body(z0, z1, x, dinv_r, w1, b1, w2, t2_o, y2_o):
    dinv = dinv_r[...]
    agg = (z0[...] + z1[...]) * dinv + x[...] * (dinv * dinv)
    h1 = jnp.dot(agg, w1[...], preferred_element_type=jnp.float32) + b1[...]
    h1 = jnp.maximum(h1, 0.0)
    t2 = jnp.dot(h1, w2[...], preferred_element_type=jnp.float32)
    t2_o[...] = t2
    y2_o[...] = t2 * dinv


def _tc2(z0, z1, x, dinv, w1, b1, w2, n_pad):
    return pl.pallas_call(
        _tc2_body,
        grid=(_row_grid(n_pad),),
        in_specs=[_rb(128), _rb(128), _rb(128), _rb(1),
                  _full((128, 256)), _full((1, 256)), _full((256, 16))],
        out_specs=[_rb(16), _rb(16)],
        out_shape=[
            jax.ShapeDtypeStruct((n_pad, 16), jnp.float32),
            jax.ShapeDtypeStruct((n_pad, 16), jnp.float32),
        ],
    )(z0, z1, x, dinv, w1, b1, w2)


def _tc3_body(z0, z1, t2, dinv_r, b2, w3, t3_o, y3_o):
    dinv = dinv_r[...]
    agg = (z0[...] + z1[...]) * dinv + t2[...] * (dinv * dinv)
    h2 = jnp.maximum(agg + b2[...], 0.0)
    t3 = jnp.dot(h2, w3[...], preferred_element_type=jnp.float32)
    t3_o[...] = t3
    y3_o[...] = t3 * dinv


def _tc3(z0, z1, t2, dinv, b2, w3, n_pad):
    return pl.pallas_call(
        _tc3_body,
        grid=(_row_grid(n_pad),),
        in_specs=[_rb(16), _rb(16), _rb(16), _rb(1),
                  _full((1, 16)), _full((16, 2))],
        out_specs=[_rb(2), _rb(2)],
        out_shape=[
            jax.ShapeDtypeStruct((n_pad, 2), jnp.float32),
            jax.ShapeDtypeStruct((n_pad, 2), jnp.float32),
        ],
    )(z0, z1, t2, dinv, b2, w3)


def _tc4_body(z0, z1, t3, dinv_r, b3, out_o):
    dinv = dinv_r[...]
    z = (z0[...] + z1[...])[:, 0:2]
    agg = z * dinv + t3[...] * (dinv * dinv)
    out_o[...] = jax.nn.sigmoid(agg + b3[...])


def _tc4(z0, z1, t3, dinv, b3, n_pad):
    return pl.pallas_call(
        _tc4_body,
        grid=(_row_grid(n_pad),),
        in_specs=[_rb(16), _rb(16), _rb(2), _rb(1), _full((1, 2))],
        out_specs=_rb(2),
        out_shape=jax.ShapeDtypeStruct((n_pad, 2), jnp.float32),
    )(z0, z1, t3, dinv, b3)


def kernel(x, edge_index, W1, b1, W2, b2, W3, b3):
    n = x.shape[0]
    d_in = x.shape[1]
    e = edge_index.shape[1]

    n_pad = ((n + 16 + 127) // 128) * 128
    e_quant = N_WORKER * CHUNK * 20
    e_pad = ((e + e_quant - 1) // e_quant) * e_quant

    src = edge_index[0].astype(jnp.int32)
    dst = edge_index[1].astype(jnp.int32)
    pad = e_pad - e
    if pad:
        pad_src = (np.arange(pad, dtype=np.int32) * 37) % n
        pad_dst = (n + (np.arange(pad, dtype=np.int32) % 16)).astype(np.int32)
        src = jnp.concatenate([src, pad_src])
        dst = jnp.concatenate([dst, pad_dst])

    zeros_wide = np.zeros((n_pad, 128), np.float32)
    zeros_16 = np.zeros((n_pad, 16), np.float32)
    ones_16 = np.ones((CHUNK, 16), np.float32)

    ck_wide = 64
    nb_wide = 5
    nb_thin = 10
    src2 = src.reshape(-1, nb_wide, ck_wide)
    dst2 = dst.reshape(-1, nb_wide, ck_wide)
    src8 = src.reshape(-1, nb_thin, CHUNK)
    dst8 = dst.reshape(-1, nb_thin, CHUNK)

    d0, d1 = _make_sc_deg(n_pad, e_pad, nb_thin)(dst8, ones_16, zeros_16)
    dinv, y1 = _tc1(d0, d1, x, n_pad, d_in)

    z0, z1 = _make_sc_agg(n_pad, d_in, e_pad, nb_wide, ck_wide, tc_tiling=True)(
        src2, dst2, y1, zeros_wide)
    t2, y2 = _tc2(z0, z1, x, dinv, W1, b1.reshape(1, -1), W2, n_pad)

    u0, u1 = _make_sc_agg(n_pad, 16, e_pad, nb_thin)(src8, dst8, y2, zeros_16)
    t3, y3 = _tc3(u0, u1, t2, dinv, b2.reshape(1, -1), W3, n_pad)

    y3_p = jnp.pad(y3, ((0, 0), (0, 14)))
    v0, v1 = _make_sc_agg(n_pad, 16, e_pad, nb_thin)(src8, dst8, y3_p, zeros_16)
    out = _tc4(v0, v1, t3, dinv, b3.reshape(1, -1), n_pad)

    return out[:n]
